# baseline (speedup 1.0000x reference)
import functools

import jax
import jax.numpy as jnp
from jax import lax
from jax.experimental import pallas as pl
from jax.experimental.pallas import tpu as pltpu
from jax.experimental.pallas import tpu_sc as plsc

N_TOK = 32768
D_IN = 768
N_EXP = 64
K_SEL = 8

NW = 32
P_CHUNKS = 4
TOK_CHUNK = N_TOK // P_CHUNKS
ROWS_PER_W = TOK_CHUNK // NW
CHUNK = 128
NCHUNK = ROWS_PER_W // CHUNK
NGRP = CHUNK // 16

MM_BLK = 2048


def _gate_body(x_ref, w1_ref, w2_ref, out_ref):
    h = jnp.tanh(lax.dot_general(
        x_ref[...], w1_ref[...], (((1,), (1,)), ((), ())),
        preferred_element_type=jnp.float32))
    out_ref[...] = lax.dot_general(
        w2_ref[...], h, (((1,), (1,)), ((), ())),
        preferred_element_type=jnp.float32)


def _gate(x, W1, W2):
    return pl.pallas_call(
        _gate_body,
        grid=(TOK_CHUNK // MM_BLK,),
        in_specs=[
            pl.BlockSpec((MM_BLK, D_IN), lambda i: (i, 0)),
            pl.BlockSpec((N_EXP, D_IN), lambda i: (0, 0)),
            pl.BlockSpec((N_EXP, N_EXP), lambda i: (0, 0)),
        ],
        out_specs=pl.BlockSpec((N_EXP, MM_BLK), lambda i: (0, i)),
        out_shape=jax.ShapeDtypeStruct((N_EXP, TOK_CHUNK), jnp.float32),
    )(x, W1, W2)


def _merge(va, ia, vb, ib):
    m = vb > va
    return jnp.where(m, vb, va), jnp.where(m, ib, ia)


def _route_body(lt_hbm, idx_hbm, scr_hbm, imp_hbm, load_hbm,
                buf0, buf1, idxb, scrb, imp2d, load2d, dsem):
    c = lax.axis_index("c")
    s = lax.axis_index("s")
    wid = s * 2 + c
    base = wid * ROWS_PER_W
    lanes = lax.iota(jnp.int32, 16)
    zeros16 = jnp.zeros((16,), jnp.float32)
    neg_inf = jnp.full((16,), -jnp.inf, jnp.float32)

    def zbody(i, carry):
        for kk in range(N_EXP // 16):
            imp2d[i, pl.ds(kk * 16, 16)] = zeros16
            load2d[i, pl.ds(kk * 16, 16)] = zeros16
        return carry

    lax.fori_loop(0, 16, zbody, 0)

    def start_in(ci, buf):
        return pltpu.async_copy(
            lt_hbm.at[:, pl.ds(base + ci * CHUNK, CHUNK)], buf, dsem)

    def process_chunk(ci, buf):
        def gbody(g, carry):
            col0 = g * 16
            rows = col0 + lanes

            bests, bidxs = [], []
            with jax.named_scope("extract"):
                blkv, blki = [], []
                for b in range(N_EXP // 8):
                    vals = [(buf[b * 8 + j, pl.ds(col0, 16)],
                             jnp.full((16,), b * 8 + j, jnp.int32))
                            for j in range(8)]
                    while len(vals) > 1:
                        vals = [_merge(*vals[p], *vals[p + 1])
                                for p in range(0, len(vals), 2)]
                    blkv.append(vals[0][0])
                    blki.append(vals[0][1])

                for r in range(K_SEL):
                    vals = list(zip(blkv, blki))
                    while len(vals) > 1:
                        vals = [_merge(*vals[p], *vals[p + 1])
                                for p in range(0, len(vals), 2)]
                    best, bidx = vals[0]
                    plsc.store_scatter(buf, [bidx, rows], neg_inf)
                    bests.append(best)
                    bidxs.append(bidx)
                    if r < K_SEL - 1:
                        wb8 = jnp.bitwise_and(bidx, jnp.int32(~7))
                        vals = [(plsc.load_gather(buf, [wb8 + j, rows]),
                                 wb8 + j)
                                for j in range(8)]
                        while len(vals) > 1:
                            vals = [_merge(*vals[p], *vals[p + 1])
                                    for p in range(0, len(vals), 2)]
                        nbv, nbi = vals[0]
                        for b in range(N_EXP // 8):
                            m = wb8 == jnp.int32(b * 8)
                            blkv[b] = jnp.where(m, nbv, blkv[b])
                            blki[b] = jnp.where(m, nbi, blki[b])

            with jax.named_scope("emit"):
                v0 = bests[0]
                es = [jnp.exp(b - v0) for b in bests]
                z = es[0]
                for e in es[1:]:
                    z = z + e
                rowl = col0 + lanes
                for r in range(K_SEL):
                    score = es[r] / z
                    rvec = jnp.full((16,), r, jnp.int32)
                    plsc.store_scatter(idxb, [rowl, rvec], bidxs[r])
                    plsc.store_scatter(scrb, [rowl, rvec], score)
                    plsc.addupdate_scatter(imp2d, [lanes, bidxs[r]], score)
                    plsc.addupdate_scatter(
                        load2d, [lanes, bidxs[r]],
                        jnp.where(score > 0, jnp.float32(1), jnp.float32(0)))
            return carry

        lax.fori_loop(0, NGRP, gbody, 0)
        row0 = base + ci * CHUNK
        pltpu.sync_copy(idxb, idx_hbm.at[pl.ds(row0, CHUNK), :])
        pltpu.sync_copy(scrb, scr_hbm.at[pl.ds(row0, CHUNK), :])

    def wait_in(buf):
        pltpu.make_async_copy(
            lt_hbm.at[:, pl.ds(base, CHUNK)], buf, dsem).wait()

    start_in(0, buf0)

    def cbody(p, carry):
        ci0 = p * 2
        wait_in(buf0)
        start_in(ci0 + 1, buf1)
        process_chunk(ci0, buf0)
        wait_in(buf1)

        @pl.when(p < NCHUNK // 2 - 1)
        def _next():
            start_in(ci0 + 2, buf0)

        process_chunk(ci0 + 1, buf1)
        return carry

    lax.fori_loop(0, NCHUNK // 2, cbody, 0)

    pltpu.sync_copy(imp2d, imp_hbm.at[pl.ds(wid * 16, 16), :])
    pltpu.sync_copy(load2d, load_hbm.at[pl.ds(wid * 16, 16), :])


def _route(lt):
    f = pl.kernel(
        _route_body,
        out_type=(
            jax.ShapeDtypeStruct((TOK_CHUNK, K_SEL), jnp.int32),
            jax.ShapeDtypeStruct((TOK_CHUNK, K_SEL), jnp.float32),
            jax.ShapeDtypeStruct((NW * 16, N_EXP), jnp.float32),
            jax.ShapeDtypeStruct((NW * 16, N_EXP), jnp.float32),
        ),
        mesh=plsc.VectorSubcoreMesh(core_axis_name="c", subcore_axis_name="s"),
        compiler_params=pltpu.CompilerParams(needs_layout_passes=False),
        scratch_types=[
            pltpu.VMEM((N_EXP, CHUNK), jnp.float32),
            pltpu.VMEM((N_EXP, CHUNK), jnp.float32),
            pltpu.VMEM((CHUNK, K_SEL), jnp.int32),
            pltpu.VMEM((CHUNK, K_SEL), jnp.float32),
            pltpu.VMEM((16, N_EXP), jnp.float32),
            pltpu.VMEM((16, N_EXP), jnp.float32),
            pltpu.SemaphoreType.DMA,
        ],
    )
    return f(lt)


def _combine_body(imp_ref, load_ref, loss_ref, load_out, imp_out):
    imp = jnp.sum(imp_ref[...], axis=0, keepdims=True)
    loadf = jnp.sum(load_ref[...], axis=0, keepdims=True)
    imp_out[...] = imp
    load_out[...] = loadf.astype(jnp.int32)

    def cv2(v):
        mean = jnp.mean(v)
        var = jnp.sum((v - mean) ** 2) / (v.size - 1)
        return var / (mean * mean + 1e-10)

    loss_ref[...] = jnp.full((1, 1), 0.01) * (cv2(imp) + cv2(loadf))


def _combine(imp_part, load_part):
    return pl.pallas_call(
        _combine_body,
        out_shape=(
            jax.ShapeDtypeStruct((1, 1), jnp.float32),
            jax.ShapeDtypeStruct((1, N_EXP), jnp.int32),
            jax.ShapeDtypeStruct((1, N_EXP), jnp.float32),
        ),
    )(imp_part, load_part)


@jax.jit
def kernel(x, W1, W2):
    idxs, scrs, imps, loads = [], [], [], []
    for p in range(P_CHUNKS):
        lt = _gate(lax.slice_in_dim(x, p * TOK_CHUNK, (p + 1) * TOK_CHUNK),
                   W1, W2)
        idx, scr, imp_part, load_part = _route(lt)
        idxs.append(idx)
        scrs.append(scr)
        imps.append(imp_part)
        loads.append(load_part)
    loss, load, imp = _combine(jnp.concatenate(imps), jnp.concatenate(loads))
    return (jnp.concatenate(idxs), jnp.concatenate(scrs), loss.reshape(()),
            load.reshape(N_EXP), imp.reshape(N_EXP))


# speedup vs baseline: 1.5730x; 1.5730x over previous
import functools

import jax
import jax.numpy as jnp
from jax import lax
from jax.experimental import pallas as pl
from jax.experimental.pallas import tpu as pltpu
from jax.experimental.pallas import tpu_sc as plsc

N_TOK = 32768
D_IN = 768
N_EXP = 64
K_SEL = 8

NW = 32
P_CHUNKS = 4
TOK_CHUNK = N_TOK // P_CHUNKS
ROWS_PER_W = TOK_CHUNK // NW
CHUNK = 128
NCHUNK = ROWS_PER_W // CHUNK
NGRP = CHUNK // 16

MM_BLK = 2048


def _gate_body(x_ref, w1_ref, w2_ref, out_ref):
    h = jnp.tanh(lax.dot_general(
        x_ref[...], w1_ref[...], (((1,), (1,)), ((), ())),
        preferred_element_type=jnp.float32))
    out_ref[...] = lax.dot_general(
        w2_ref[...], h, (((1,), (1,)), ((), ())),
        preferred_element_type=jnp.float32)


def _gate(x, W1, W2, p):
    off = p * (TOK_CHUNK // MM_BLK)
    return pl.pallas_call(
        _gate_body,
        grid=(TOK_CHUNK // MM_BLK,),
        in_specs=[
            pl.BlockSpec((MM_BLK, D_IN), lambda i: (i + off, 0)),
            pl.BlockSpec((N_EXP, D_IN), lambda i: (0, 0)),
            pl.BlockSpec((N_EXP, N_EXP), lambda i: (0, 0)),
        ],
        out_specs=pl.BlockSpec((N_EXP, MM_BLK), lambda i: (0, i)),
        out_shape=jax.ShapeDtypeStruct((N_EXP, TOK_CHUNK), jnp.float32),
    )(x, W1, W2)


def _merge(va, ia, vb, ib):
    m = vb > va
    return jnp.where(m, vb, va), jnp.where(m, ib, ia)


def _route_body(lt_hbm, idx_hbm, scr_hbm, imp_hbm, load_hbm,
                buf0, buf1, idxb, scrb, imp2d, load2d, dsem):
    c = lax.axis_index("c")
    s = lax.axis_index("s")
    wid = s * 2 + c
    base = wid * ROWS_PER_W
    lanes = lax.iota(jnp.int32, 16)
    zeros16 = jnp.zeros((16,), jnp.float32)
    neg_inf = jnp.full((16,), -jnp.inf, jnp.float32)

    def zbody(i, carry):
        for kk in range(N_EXP // 16):
            imp2d[i, pl.ds(kk * 16, 16)] = zeros16
            load2d[i, pl.ds(kk * 16, 16)] = zeros16
        return carry

    lax.fori_loop(0, 16, zbody, 0)

    def start_in(ci, buf):
        return pltpu.async_copy(
            lt_hbm.at[:, pl.ds(base + ci * CHUNK, CHUNK)], buf, dsem)

    def process_chunk(ci, buf):
        def gbody(g, carry):
            col0 = g * 16
            rows = col0 + lanes

            bests, bidxs = [], []
            with jax.named_scope("extract"):
                blkv, blki = [], []
                for b in range(N_EXP // 8):
                    vals = [(buf[b * 8 + j, pl.ds(col0, 16)],
                             jnp.full((16,), b * 8 + j, jnp.int32))
                            for j in range(8)]
                    while len(vals) > 1:
                        vals = [_merge(*vals[p], *vals[p + 1])
                                for p in range(0, len(vals), 2)]
                    blkv.append(vals[0][0])
                    blki.append(vals[0][1])

                for r in range(K_SEL):
                    vals = list(zip(blkv, blki))
                    while len(vals) > 1:
                        vals = [_merge(*vals[p], *vals[p + 1])
                                for p in range(0, len(vals), 2)]
                    best, bidx = vals[0]
                    plsc.store_scatter(buf, [bidx, rows], neg_inf)
                    bests.append(best)
                    bidxs.append(bidx)
                    if r < K_SEL - 1:
                        wb8 = jnp.bitwise_and(bidx, jnp.int32(~7))
                        vals = [(plsc.load_gather(buf, [wb8 + j, rows]),
                                 wb8 + j)
                                for j in range(8)]
                        while len(vals) > 1:
                            vals = [_merge(*vals[p], *vals[p + 1])
                                    for p in range(0, len(vals), 2)]
                        nbv, nbi = vals[0]
                        for b in range(N_EXP // 8):
                            m = wb8 == jnp.int32(b * 8)
                            blkv[b] = jnp.where(m, nbv, blkv[b])
                            blki[b] = jnp.where(m, nbi, blki[b])

            with jax.named_scope("emit"):
                v0 = bests[0]
                es = [jnp.exp(b - v0) for b in bests]
                z = es[0]
                for e in es[1:]:
                    z = z + e
                rowl = col0 + lanes
                for r in range(K_SEL):
                    score = es[r] / z
                    rvec = jnp.full((16,), r, jnp.int32)
                    plsc.store_scatter(idxb, [rowl, rvec], bidxs[r])
                    plsc.store_scatter(scrb, [rowl, rvec], score)
                    plsc.addupdate_scatter(imp2d, [lanes, bidxs[r]], score)
                    plsc.addupdate_scatter(
                        load2d, [lanes, bidxs[r]],
                        jnp.where(score > 0, jnp.float32(1), jnp.float32(0)))
            return carry

        lax.fori_loop(0, NGRP, gbody, 0)
        row0 = base + ci * CHUNK
        pltpu.sync_copy(idxb, idx_hbm.at[pl.ds(row0, CHUNK), :])
        pltpu.sync_copy(scrb, scr_hbm.at[pl.ds(row0, CHUNK), :])

    def wait_in(buf):
        pltpu.make_async_copy(
            lt_hbm.at[:, pl.ds(base, CHUNK)], buf, dsem).wait()

    start_in(0, buf0)

    def cbody(p, carry):
        ci0 = p * 2
        wait_in(buf0)
        start_in(ci0 + 1, buf1)
        process_chunk(ci0, buf0)
        wait_in(buf1)

        @pl.when(p < NCHUNK // 2 - 1)
        def _next():
            start_in(ci0 + 2, buf0)

        process_chunk(ci0 + 1, buf1)
        return carry

    lax.fori_loop(0, NCHUNK // 2, cbody, 0)

    pltpu.sync_copy(imp2d, imp_hbm.at[pl.ds(wid * 16, 16), :])
    pltpu.sync_copy(load2d, load_hbm.at[pl.ds(wid * 16, 16), :])


def _route(lt):
    f = pl.kernel(
        _route_body,
        out_type=(
            jax.ShapeDtypeStruct((TOK_CHUNK, K_SEL), jnp.int32),
            jax.ShapeDtypeStruct((TOK_CHUNK, K_SEL), jnp.float32),
            jax.ShapeDtypeStruct((NW * 16, N_EXP), jnp.float32),
            jax.ShapeDtypeStruct((NW * 16, N_EXP), jnp.float32),
        ),
        mesh=plsc.VectorSubcoreMesh(core_axis_name="c", subcore_axis_name="s"),
        compiler_params=pltpu.CompilerParams(needs_layout_passes=False),
        scratch_types=[
            pltpu.VMEM((N_EXP, CHUNK), jnp.float32),
            pltpu.VMEM((N_EXP, CHUNK), jnp.float32),
            pltpu.VMEM((CHUNK, K_SEL), jnp.int32),
            pltpu.VMEM((CHUNK, K_SEL), jnp.float32),
            pltpu.VMEM((16, N_EXP), jnp.float32),
            pltpu.VMEM((16, N_EXP), jnp.float32),
            pltpu.SemaphoreType.DMA,
        ],
    )
    return f(lt)


def _combine_body(imp_ref, load_ref, loss_ref, load_out, imp_out):
    imp = jnp.sum(imp_ref[...], axis=0, keepdims=True)
    loadf = jnp.sum(load_ref[...], axis=0, keepdims=True)
    imp_out[...] = imp
    load_out[...] = loadf.astype(jnp.int32)

    def cv2(v):
        mean = jnp.mean(v)
        var = jnp.sum((v - mean) ** 2) / (v.size - 1)
        return var / (mean * mean + 1e-10)

    loss_ref[...] = jnp.full((1, 1), 0.01) * (cv2(imp) + cv2(loadf))


def _combine(imp_part, load_part):
    return pl.pallas_call(
        _combine_body,
        out_shape=(
            jax.ShapeDtypeStruct((1, 1), jnp.float32),
            jax.ShapeDtypeStruct((1, N_EXP), jnp.int32),
            jax.ShapeDtypeStruct((1, N_EXP), jnp.float32),
        ),
    )(imp_part, load_part)


@jax.jit
def kernel(x, W1, W2):
    idxs, scrs, imps, loads = [], [], [], []
    for p in range(P_CHUNKS):
        lt = _gate(x, W1, W2, p)
        idx, scr, imp_part, load_part = _route(lt)
        idxs.append(idx)
        scrs.append(scr)
        imps.append(imp_part)
        loads.append(load_part)
    loss, load, imp = _combine(jnp.concatenate(imps), jnp.concatenate(loads))
    return (jnp.concatenate(idxs), jnp.concatenate(scrs), loss.reshape(()),
            load.reshape(N_EXP), imp.reshape(N_EXP))


# speedup vs baseline: 1.6803x; 1.0683x over previous
import functools

import jax
import jax.numpy as jnp
from jax import lax
from jax.experimental import pallas as pl
from jax.experimental.pallas import tpu as pltpu
from jax.experimental.pallas import tpu_sc as plsc

N_TOK = 32768
D_IN = 768
N_EXP = 64
K_SEL = 8

NW = 32
P_CHUNKS = 2
TOK_CHUNK = N_TOK // P_CHUNKS
ROWS_PER_W = TOK_CHUNK // NW
CHUNK = 128
NCHUNK = ROWS_PER_W // CHUNK
NGRP = CHUNK // 16

MM_BLK = 2048


def _gate_body(x_ref, w1_ref, w2_ref, out_ref):
    h = jnp.tanh(lax.dot_general(
        x_ref[...], w1_ref[...], (((1,), (1,)), ((), ())),
        preferred_element_type=jnp.float32))
    out_ref[...] = lax.dot_general(
        w2_ref[...], h, (((1,), (1,)), ((), ())),
        preferred_element_type=jnp.float32)


def _gate(x, W1, W2, p):
    off = p * (TOK_CHUNK // MM_BLK)
    return pl.pallas_call(
        _gate_body,
        grid=(TOK_CHUNK // MM_BLK,),
        in_specs=[
            pl.BlockSpec((MM_BLK, D_IN), lambda i: (i + off, 0)),
            pl.BlockSpec((N_EXP, D_IN), lambda i: (0, 0)),
            pl.BlockSpec((N_EXP, N_EXP), lambda i: (0, 0)),
        ],
        out_specs=pl.BlockSpec((N_EXP, MM_BLK), lambda i: (0, i)),
        out_shape=jax.ShapeDtypeStruct((N_EXP, TOK_CHUNK), jnp.float32),
    )(x, W1, W2)


def _merge(va, ia, vb, ib):
    m = vb > va
    return jnp.where(m, vb, va), jnp.where(m, ib, ia)


def _route_body(lt_hbm, idx_hbm, scr_hbm, imp_hbm, load_hbm,
                buf0, buf1, idxb, scrb, imp2d, load2d, dsem):
    c = lax.axis_index("c")
    s = lax.axis_index("s")
    wid = s * 2 + c
    base = wid * ROWS_PER_W
    lanes = lax.iota(jnp.int32, 16)
    zeros16 = jnp.zeros((16,), jnp.float32)
    neg_inf = jnp.full((16,), -jnp.inf, jnp.float32)

    def zbody(i, carry):
        for kk in range(N_EXP // 16):
            imp2d[i, pl.ds(kk * 16, 16)] = zeros16
            load2d[i, pl.ds(kk * 16, 16)] = zeros16
        return carry

    lax.fori_loop(0, 16, zbody, 0)

    def start_in(ci, buf):
        return pltpu.async_copy(
            lt_hbm.at[:, pl.ds(base + ci * CHUNK, CHUNK)], buf, dsem)

    def process_chunk(ci, buf):
        def gbody(g, carry):
            col0 = g * 16
            rows = col0 + lanes

            bests, bidxs = [], []
            with jax.named_scope("extract"):
                blkv, blki = [], []
                for b in range(N_EXP // 8):
                    vals = [(buf[b * 8 + j, pl.ds(col0, 16)],
                             jnp.full((16,), b * 8 + j, jnp.int32))
                            for j in range(8)]
                    while len(vals) > 1:
                        vals = [_merge(*vals[p], *vals[p + 1])
                                for p in range(0, len(vals), 2)]
                    blkv.append(vals[0][0])
                    blki.append(vals[0][1])

                for r in range(K_SEL):
                    vals = list(zip(blkv, blki))
                    while len(vals) > 1:
                        vals = [_merge(*vals[p], *vals[p + 1])
                                for p in range(0, len(vals), 2)]
                    best, bidx = vals[0]
                    plsc.store_scatter(buf, [bidx, rows], neg_inf)
                    bests.append(best)
                    bidxs.append(bidx)
                    if r < K_SEL - 1:
                        wb8 = jnp.bitwise_and(bidx, jnp.int32(~7))
                        vals = [(plsc.load_gather(buf, [wb8 + j, rows]),
                                 wb8 + j)
                                for j in range(8)]
                        while len(vals) > 1:
                            vals = [_merge(*vals[p], *vals[p + 1])
                                    for p in range(0, len(vals), 2)]
                        nbv, nbi = vals[0]
                        for b in range(N_EXP // 8):
                            m = wb8 == jnp.int32(b * 8)
                            blkv[b] = jnp.where(m, nbv, blkv[b])
                            blki[b] = jnp.where(m, nbi, blki[b])

            with jax.named_scope("emit"):
                v0 = bests[0]
                es = [jnp.exp(b - v0) for b in bests]
                z = es[0]
                for e in es[1:]:
                    z = z + e
                rowl = col0 + lanes
                for r in range(K_SEL):
                    score = es[r] / z
                    rvec = jnp.full((16,), r, jnp.int32)
                    plsc.store_scatter(idxb, [rowl, rvec], bidxs[r])
                    plsc.store_scatter(scrb, [rowl, rvec], score)
                    plsc.addupdate_scatter(imp2d, [lanes, bidxs[r]], score)
                    plsc.addupdate_scatter(
                        load2d, [lanes, bidxs[r]],
                        jnp.where(score > 0, jnp.float32(1), jnp.float32(0)))
            return carry

        lax.fori_loop(0, NGRP, gbody, 0)
        row0 = base + ci * CHUNK
        pltpu.sync_copy(idxb, idx_hbm.at[pl.ds(row0, CHUNK), :])
        pltpu.sync_copy(scrb, scr_hbm.at[pl.ds(row0, CHUNK), :])

    def wait_in(buf):
        pltpu.make_async_copy(
            lt_hbm.at[:, pl.ds(base, CHUNK)], buf, dsem).wait()

    start_in(0, buf0)

    def cbody(p, carry):
        ci0 = p * 2
        wait_in(buf0)
        start_in(ci0 + 1, buf1)
        process_chunk(ci0, buf0)
        wait_in(buf1)

        @pl.when(p < NCHUNK // 2 - 1)
        def _next():
            start_in(ci0 + 2, buf0)

        process_chunk(ci0 + 1, buf1)
        return carry

    lax.fori_loop(0, NCHUNK // 2, cbody, 0)

    pltpu.sync_copy(imp2d, imp_hbm.at[pl.ds(wid * 16, 16), :])
    pltpu.sync_copy(load2d, load_hbm.at[pl.ds(wid * 16, 16), :])


def _route(lt):
    f = pl.kernel(
        _route_body,
        out_type=(
            jax.ShapeDtypeStruct((TOK_CHUNK, K_SEL), jnp.int32),
            jax.ShapeDtypeStruct((TOK_CHUNK, K_SEL), jnp.float32),
            jax.ShapeDtypeStruct((NW * 16, N_EXP), jnp.float32),
            jax.ShapeDtypeStruct((NW * 16, N_EXP), jnp.float32),
        ),
        mesh=plsc.VectorSubcoreMesh(core_axis_name="c", subcore_axis_name="s"),
        compiler_params=pltpu.CompilerParams(needs_layout_passes=False),
        scratch_types=[
            pltpu.VMEM((N_EXP, CHUNK), jnp.float32),
            pltpu.VMEM((N_EXP, CHUNK), jnp.float32),
            pltpu.VMEM((CHUNK, K_SEL), jnp.int32),
            pltpu.VMEM((CHUNK, K_SEL), jnp.float32),
            pltpu.VMEM((16, N_EXP), jnp.float32),
            pltpu.VMEM((16, N_EXP), jnp.float32),
            pltpu.SemaphoreType.DMA,
        ],
    )
    return f(lt)


def _combine_body(imp_ref, load_ref, loss_ref, load_out, imp_out):
    imp = jnp.sum(imp_ref[...], axis=0, keepdims=True)
    loadf = jnp.sum(load_ref[...], axis=0, keepdims=True)
    imp_out[...] = imp
    load_out[...] = loadf.astype(jnp.int32)

    def cv2(v):
        mean = jnp.mean(v)
        var = jnp.sum((v - mean) ** 2) / (v.size - 1)
        return var / (mean * mean + 1e-10)

    loss_ref[...] = jnp.full((1, 1), 0.01) * (cv2(imp) + cv2(loadf))


def _combine(imp_part, load_part):
    return pl.pallas_call(
        _combine_body,
        out_shape=(
            jax.ShapeDtypeStruct((1, 1), jnp.float32),
            jax.ShapeDtypeStruct((1, N_EXP), jnp.int32),
            jax.ShapeDtypeStruct((1, N_EXP), jnp.float32),
        ),
    )(imp_part, load_part)


@jax.jit
def kernel(x, W1, W2):
    idxs, scrs, imps, loads = [], [], [], []
    for p in range(P_CHUNKS):
        lt = _gate(x, W1, W2, p)
        idx, scr, imp_part, load_part = _route(lt)
        idxs.append(idx)
        scrs.append(scr)
        imps.append(imp_part)
        loads.append(load_part)
    loss, load, imp = _combine(jnp.concatenate(imps), jnp.concatenate(loads))
    return (jnp.concatenate(idxs), jnp.concatenate(scrs), loss.reshape(()),
            load.reshape(N_EXP), imp.reshape(N_EXP))
